# baseline (speedup 1.0000x reference)
import functools

import jax
import jax.numpy as jnp
from jax import lax
from jax.experimental import pallas as pl
from jax.experimental.pallas import tpu as pltpu
from jax.experimental.pallas import tpu_sc as plsc

B = 16384
D = 64
NC = 2
NS = 16
NW = NC * NS
BPW = B // NW
IDXW = 128
NCHUNK = BPW // IDXW
LANES = 16

_mesh = plsc.VectorSubcoreMesh(core_axis_name="c", subcore_axis_name="s")


@functools.partial(
    pl.kernel,
    mesh=_mesh,
    compiler_params=pltpu.CompilerParams(use_tc_tiling_on_sc=False),
    out_type=jax.ShapeDtypeStruct((B, D), jnp.float32),
    scratch_types=[
        pltpu.VMEM((NCHUNK, IDXW), jnp.int32),
        pltpu.VMEM((BPW, D), jnp.float32),
        pltpu.VMEM((BPW, D), jnp.float32),
        pltpu.SemaphoreType.DMA,
    ],
)
def _embed_add(ctx_hbm, labels_hbm, table_hbm, out_hbm, idx_v, rows_v, ctx_v, sem):
    wid = lax.axis_index("s") * NC + lax.axis_index("c")
    base = wid * BPW

    pltpu.sync_copy(labels_hbm.at[wid], idx_v)

    copies = [
        pltpu.async_copy(
            table_hbm.at[idx_v.at[j]], rows_v.at[pl.ds(j * IDXW, IDXW)], sem
        )
        for j in range(NCHUNK)
    ]
    pltpu.sync_copy(ctx_hbm.at[pl.ds(base, BPW)], ctx_v)
    for c in copies:
        c.wait()

    def body(r, carry):
        for c in range(D // LANES):
            sl = pl.ds(c * LANES, LANES)
            rows_v[r, sl] = rows_v[r, sl] + ctx_v[r, sl]
        return carry

    lax.fori_loop(0, BPW, body, 0)

    pltpu.sync_copy(rows_v, out_hbm.at[pl.ds(base, BPW)])


def kernel(ctx_vec, labels, emb_weight):
    labels_r = labels.astype(jnp.int32).reshape(NW, NCHUNK, IDXW)
    return _embed_add(ctx_vec, labels_r, emb_weight)
